# Initial kernel scaffold; baseline (speedup 1.0000x reference)
import jax
import jax.numpy as jnp
from jax.experimental import pallas as pl


def kernel(x, edge_index, W1, b1, W2, b2):
    raise NotImplementedError("write your pallas kernel here")


# speedup vs baseline: 18.0390x; 18.0390x over previous
import functools

import jax
import jax.numpy as jnp
from jax import lax
from jax.experimental import pallas as pl
from jax.experimental.pallas import tpu as pltpu
from jax.experimental.pallas import tpu_sc as plsc

N_NODES = 10000
IN_DIM = 128
HID_DIM = 128
OUT_DIM = 16

NC = 2
NS = 16
NW = NC * NS
K = 128
STRIPE = 640
NPAD = NS * STRIPE
ZR = 128
BR = 1000


def _fill(ref, rows, cols, value):
    groups = cols // 16

    def body(t, carry):
        r = t // groups
        g = t % groups
        ref[r, pl.ds(g * 16, 16)] = jnp.full((16,), value, jnp.float32)
        return carry

    lax.fori_loop(0, rows * groups, body, 0)


def _make_deg_kernel(n_blocks):
    mesh = plsc.VectorSubcoreMesh(core_axis_name="c", subcore_axis_name="s")

    @functools.partial(
        pl.kernel,
        mesh=mesh,
        out_type=jax.ShapeDtypeStruct((NC, NPAD, 16), jnp.float32),
        scratch_types=[
            pltpu.VMEM((n_blocks, K), jnp.int32),
            pltpu.VMEM((K, 16), jnp.float32),
            pltpu.VMEM_SHARED((NPAD, 16), jnp.float32),
        ],
        compiler_params=pltpu.CompilerParams(use_tc_tiling_on_sc=False),
    )
    def deg_kernel(dst_hbm, out_hbm, dst_v, ones_v, acc):
        c = lax.axis_index("c")
        s = lax.axis_index("s")
        wid = c * NS + s
        _fill(ones_v, K, 16, 0.0)
        for kk in range(STRIPE // ZR):
            pltpu.sync_copy(ones_v, acc.at[pl.ds(s * STRIPE + kk * ZR, ZR)])
        _fill(ones_v, K, 16, 1.0)
        pltpu.sync_copy(dst_hbm.at[wid], dst_v)
        plsc.subcore_barrier()

        def body(j, carry):
            pltpu.sync_copy(ones_v, acc.at[dst_v.at[j]], add=True)
            return carry

        lax.fori_loop(0, n_blocks, body, 0)
        plsc.subcore_barrier()
        for kk in range(STRIPE // ZR):
            off = s * STRIPE + kk * ZR
            pltpu.sync_copy(acc.at[pl.ds(off, ZR)], out_hbm.at[c, pl.ds(off, ZR)])

    return deg_kernel


def _make_agg_kernel(n_blocks, d):
    mesh = plsc.VectorSubcoreMesh(core_axis_name="c", subcore_axis_name="s")

    @functools.partial(
        pl.kernel,
        mesh=mesh,
        out_type=jax.ShapeDtypeStruct((NC, NPAD, d), jnp.float32),
        scratch_types=[
            pltpu.VMEM((n_blocks, K), jnp.int32),
            pltpu.VMEM((n_blocks, K), jnp.int32),
            pltpu.VMEM((K, d), jnp.float32),
            pltpu.VMEM_SHARED((NPAD, d), jnp.float32),
            pltpu.SemaphoreType.DMA,
        ],
        compiler_params=pltpu.CompilerParams(use_tc_tiling_on_sc=False),
    )
    def agg_kernel(z_hbm, src_hbm, dst_hbm, out_hbm, src_v, dst_v, buf, acc, sem):
        c = lax.axis_index("c")
        s = lax.axis_index("s")
        wid = c * NS + s
        _fill(buf, K, d, 0.0)
        for kk in range(STRIPE // ZR):
            pltpu.sync_copy(buf, acc.at[pl.ds(s * STRIPE + kk * ZR, ZR)])
        pltpu.sync_copy(src_hbm.at[wid], src_v)
        pltpu.sync_copy(dst_hbm.at[wid], dst_v)
        plsc.subcore_barrier()

        def body(j, carry):
            pltpu.async_copy(z_hbm.at[src_v.at[j]], buf, sem).wait()
            pltpu.sync_copy(buf, acc.at[dst_v.at[j]], add=True)
            return carry

        lax.fori_loop(0, n_blocks, body, 0)
        plsc.subcore_barrier()
        for kk in range(STRIPE // ZR):
            off = s * STRIPE + kk * ZR
            pltpu.sync_copy(acc.at[pl.ds(off, ZR)], out_hbm.at[c, pl.ds(off, ZR)])

    return agg_kernel


def _dinv_of(d0, d1):
    return lax.rsqrt(1.0 + d0[:, :1] + d1[:, :1])


def _mm1_body(x_ref, d0_ref, d1_ref, w_ref, o_ref):
    dinv = _dinv_of(d0_ref[...], d1_ref[...])
    y = jnp.dot(x_ref[...], w_ref[...], preferred_element_type=jnp.float32)
    o_ref[...] = y * dinv


def _fuse_body(p0_ref, p1_ref, z1_ref, d0_ref, d1_ref, b1_ref, w2_ref, o_ref):
    dinv = _dinv_of(d0_ref[...], d1_ref[...])
    h = dinv * (p0_ref[...] + p1_ref[...] + z1_ref[...]) + b1_ref[...]
    h = jnp.maximum(h, 0.0)
    y2 = jnp.dot(h, w2_ref[...], preferred_element_type=jnp.float32)
    o_ref[...] = y2 * dinv


def _fin_body(q0_ref, q1_ref, z2_ref, d0_ref, d1_ref, b2_ref, o_ref):
    dinv = _dinv_of(d0_ref[...], d1_ref[...])
    o_ref[...] = dinv * (q0_ref[...] + q1_ref[...] + z2_ref[...]) + b2_ref[...]


def _row_spec(cols):
    return pl.BlockSpec((BR, cols), lambda i: (i, 0))


def _full_spec(rows, cols):
    return pl.BlockSpec((rows, cols), lambda i: (0, 0))


def kernel(x, edge_index, W1, b1, W2, b2):
    src = edge_index[0].astype(jnp.int32)
    dst = edge_index[1].astype(jnp.int32)
    e = src.shape[0]
    n_blocks = -(-e // (NW * K))
    pad = NW * n_blocks * K - e
    src_p = jnp.concatenate([src, jnp.zeros((pad,), jnp.int32)]).reshape(NW, n_blocks, K)
    dst_p = jnp.concatenate([dst, jnp.full((pad,), N_NODES, jnp.int32)]).reshape(NW, n_blocks, K)

    degp = _make_deg_kernel(n_blocks)(dst_p)
    d0 = degp[0, :N_NODES]
    d1 = degp[1, :N_NODES]

    grid = (N_NODES // BR,)
    z1 = pl.pallas_call(
        _mm1_body,
        grid=grid,
        in_specs=[_row_spec(IN_DIM), _row_spec(16), _row_spec(16),
                  _full_spec(IN_DIM, HID_DIM)],
        out_specs=_row_spec(HID_DIM),
        out_shape=jax.ShapeDtypeStruct((N_NODES, HID_DIM), jnp.float32),
    )(x, d0, d1, W1)

    agg1 = _make_agg_kernel(n_blocks, HID_DIM)(z1, src_p, dst_p)

    z2 = pl.pallas_call(
        _fuse_body,
        grid=grid,
        in_specs=[_row_spec(HID_DIM), _row_spec(HID_DIM), _row_spec(HID_DIM),
                  _row_spec(16), _row_spec(16),
                  _full_spec(1, HID_DIM), _full_spec(HID_DIM, OUT_DIM)],
        out_specs=_row_spec(OUT_DIM),
        out_shape=jax.ShapeDtypeStruct((N_NODES, OUT_DIM), jnp.float32),
    )(agg1[0, :N_NODES], agg1[1, :N_NODES], z1, d0, d1,
      b1.reshape(1, HID_DIM), W2)

    agg2 = _make_agg_kernel(n_blocks, OUT_DIM)(z2, src_p, dst_p)

    out = pl.pallas_call(
        _fin_body,
        grid=grid,
        in_specs=[_row_spec(OUT_DIM), _row_spec(OUT_DIM), _row_spec(OUT_DIM),
                  _row_spec(16), _row_spec(16), _full_spec(1, OUT_DIM)],
        out_specs=_row_spec(OUT_DIM),
        out_shape=jax.ShapeDtypeStruct((N_NODES, OUT_DIM), jnp.float32),
    )(agg2[0, :N_NODES], agg2[1, :N_NODES], z2, d0, d1, b2.reshape(1, OUT_DIM))

    return out
